# baseline (speedup 1.0000x reference)
import functools

import numpy as np
import jax
import jax.numpy as jnp
from jax.experimental import pallas as pl
from jax.experimental.pallas import tpu as pltpu
from jax.experimental.pallas import tpu_sc as plsc

B = 4; N_AB = 1024; N_AG = 1536; NPC = N_AB + N_AG; N = B * NPC
C_ATOM = 4; CA = 1; K_NEI = 9; EMB = 64; HID = 128; KP = 10; L = 4; RBF = 16
ORDERS = (2, 3, 4, 5, 6); NORD = len(ORDERS)
NODE_IN = 3 * EMB; MSG_IN = 2 * HID + RBF + NORD; STD = 10.0
_TRI = np.array([(a, b, c) for a in range(KP) for b in range(a + 1, KP) for c in range(b + 1, KP)], dtype=np.int32)
_CENTERS = jnp.linspace(0.0, 1.5, RBF)

BLK = 512
NBLK = N // BLK
NBLK_H = (N // 4) // BLK

F32 = jnp.float32


def _mse(a, b):
    return jnp.mean((a - b) ** 2)


def _cdist(a, b):
    return jnp.sqrt(jnp.sum((a[:, None, :] - b[None, :, :]) ** 2, -1) + 1e-12)


def _rbf_feat(d):
    return jnp.exp(-((d[:, None] - _CENTERS[None, :]) ** 2) / (2 * 0.1 ** 2))


def _rots():
    A = jax.random.normal(jax.random.key(42), (B, 3, 3))
    Q, _ = jnp.linalg.qr(A)
    d = jnp.sign(jnp.linalg.det(Q))
    Q = Q.at[:, :, 2].multiply(d[:, None])
    tr = jax.random.uniform(jax.random.key(43), (B, 3))
    return Q, tr


_ROTS_C, _TR_C = (np.asarray(v) for v in jax.jit(_rots)())


RBLK = 512
NRBLK = NPC // RBLK


def _knn_kernel(xr_ref, xct_ref, nbr_ref, d2_ref):
    b = pl.program_id(0)
    rb = pl.program_id(1)
    xr = xr_ref[0]
    xct = xct_ref[0]
    sqr = jnp.sum(xr * xr, -1, keepdims=True)
    sqc = jnp.sum(xct * xct, 0, keepdims=True)
    d2 = sqr + sqc - 2.0 * jnp.dot(xr, xct, preferred_element_type=F32, precision=jax.lax.Precision.HIGHEST)
    row0 = rb * RBLK
    riota = jax.lax.broadcasted_iota(jnp.int32, (RBLK, NPC), 0) + row0
    ciota = jax.lax.broadcasted_iota(jnp.int32, (RBLK, NPC), 1)
    d2_ref[...] = jnp.where(riota == ciota, d2 + 1e9, d2)
    cols = []
    for _ in range(K_NEI):
        d2m = d2_ref[...]
        mn = jnp.min(d2m, axis=1, keepdims=True)
        cand = jnp.where(d2m == mn, ciota, jnp.int32(NPC + 1))
        idx = jnp.min(cand, axis=1, keepdims=True)
        cols.append(idx)
        d2_ref[...] = jnp.where(ciota == idx, jnp.float32(jnp.inf), d2m)
    nbr_ref[...] = jnp.concatenate(cols, axis=1)


def _knn(Xca):
    Xb = Xca.reshape(B, NPC, 3)
    XbT = Xb.transpose(0, 2, 1)
    col = pl.pallas_call(
        _knn_kernel,
        grid=(B, NRBLK),
        in_specs=[
            pl.BlockSpec((1, RBLK, 3), lambda b, r: (b, r, 0)),
            pl.BlockSpec((1, 3, NPC), lambda b, r: (b, 0, 0)),
        ],
        out_specs=pl.BlockSpec((RBLK, K_NEI), lambda b, r: (b * NRBLK + r, 0)),
        out_shape=jax.ShapeDtypeStruct((N, K_NEI), jnp.int32),
        scratch_shapes=[pltpu.VMEM((RBLK, NPC), F32)],
        compiler_params=pltpu.CompilerParams(dimension_semantics=("parallel", "parallel")),
    )(Xb, XbT)
    return col


NH = N // 4
E_TOT = NH * K_NEI
DG = 256
_SC = plsc.get_sparse_core_info()
NW = _SC.num_cores * _SC.num_subcores
EPW = E_TOT // NW
CH = 240
NCH = EPW // CH


@functools.partial(
    pl.kernel,
    mesh=plsc.VectorSubcoreMesh(core_axis_name="c", subcore_axis_name="s"),
    out_type=jax.ShapeDtypeStruct((E_TOT, DG), jnp.float32),
    scratch_types=[
        pltpu.VMEM((CH,), jnp.int32),
        pltpu.VMEM((CH,), jnp.int32),
        pltpu.VMEM((CH, DG), jnp.float32),
        pltpu.VMEM((CH, DG), jnp.float32),
        pltpu.SemaphoreType.DMA,
    ],
)
def _sc_gather(table_hbm, idx_hbm, out_hbm, idx0, idx1, rows0, rows1, sem):
    wid = jax.lax.axis_index("s") * _SC.num_cores + jax.lax.axis_index("c")
    base = wid * EPW
    idxb = [idx0, idx1]
    rowsb = [rows0, rows1]
    pltpu.sync_copy(idx_hbm.at[pl.ds(base, CH)], idxb[0])
    copies = [pltpu.async_copy(table_hbm.at[idxb[0]], rowsb[0], sem)]
    for i in range(NCH):
        if i + 1 < NCH:
            pltpu.sync_copy(idx_hbm.at[pl.ds(base + (i + 1) * CH, CH)], idxb[(i + 1) % 2])
            copies.append(pltpu.async_copy(table_hbm.at[idxb[(i + 1) % 2]], rowsb[(i + 1) % 2], sem))
        copies[i].wait()
        pltpu.sync_copy(rowsb[i % 2], out_hbm.at[pl.ds(base + i * CH, CH)])


def _gather_hx(state_h, idx_h):
    return _sc_gather(state_h, idx_h).reshape(K_NEI, NH, DG)


def _kabsch(Y1, Y2):
    c1 = Y1.mean(0); c2 = Y2.mean(0)
    Hm = (Y1 - c1).T @ (Y2 - c2)
    U, _, Vt = jnp.linalg.svd(Hm, full_matrices=False)
    d = jnp.sign(jnp.linalg.det(U @ Vt))
    D = jnp.diag(jnp.concatenate([jnp.ones(2), d[None]]))
    R = U @ D @ Vt
    t = c2 - c1 @ R
    return R, t


def _maxtri(Y):
    a = Y[_TRI[:, 0]]; b = Y[_TRI[:, 1]]; c = Y[_TRI[:, 2]]
    return jnp.max(0.5 * jnp.linalg.norm(jnp.cross(b - a, c - a), axis=-1))


def _layer_body(first, s_ref, hcx_ref, ea_ref, nv_ref, na_ref,
                we1a, we1b, we1c, we1d, be1, we2, be2, waT, ba,
                wx1, bx1, wx2T, bx2, wh1a, wh1b, wh1c, bh1, wh2, bh2,
                sout_ref, eaout_ref=None, nvout_ref=None, cen_ref=None):
    sblk = s_ref[...]
    h = sblk[:, :HID]
    x = sblk[:, HID:HID + 3]
    if first:
        ea_js = []
        fo = [jnp.zeros((BLK, 3), F32) for _ in ORDERS]
        for j in range(K_NEI):
            xc = hcx_ref[j][:, HID:HID + 3]
            dj = x - xc
            d0 = jnp.sqrt(jnp.sum(dj * dj, -1, keepdims=True))
            ea_j = jnp.exp(-((d0 - cen_ref[...]) ** 2) / (2 * 0.1 ** 2))
            ea_js.append(ea_j)
            eaout_ref[j] = ea_j
            dist = d0 + 1e-8
            for o in range(NORD):
                fo[o] = fo[o] + dj / dist ** ORDERS[o]
        nv_js = []
        for o in range(NORD):
            f = fo[o]
            nv_o = f / (jnp.sqrt(jnp.sum(f * f, -1, keepdims=True)) + 1e-8)
            nv_js.append(nv_o)
            nvout_ref[o] = nv_o
    base = jnp.dot(h, we1a[...], preferred_element_type=F32, precision=jax.lax.Precision.HIGHEST) + be1[...]
    agg = jnp.zeros((BLK, HID), F32)
    xdelta = jnp.zeros((BLK, 3), F32)
    for j in range(K_NEI):
        hcx = hcx_ref[j]
        hc = hcx[:, :HID]
        xc = hcx[:, HID:HID + 3]
        d = x - xc
        dist = jnp.sqrt(jnp.sum(d * d, -1, keepdims=True)) + 1e-8
        u = d / dist
        ea_j = ea_js[j] if first else ea_ref[j]
        mpre = base + jnp.dot(hc, we1b[...], preferred_element_type=F32, precision=jax.lax.Precision.HIGHEST)
        mpre = mpre + jnp.dot(ea_j, we1c[...], preferred_element_type=F32, precision=jax.lax.Precision.HIGHEST)
        for o in range(NORD):
            nv_o = nv_js[o] if first else nv_ref[o]
            nf_o = jnp.sum(nv_o * u, -1, keepdims=True)
            mpre = mpre + nf_o * we1d[o:o + 1, :]
        m = jax.nn.silu(mpre)
        m = jax.nn.silu(jnp.dot(m, we2[...], preferred_element_type=F32, precision=jax.lax.Precision.HIGHEST) + be2[...])
        att = jax.nn.sigmoid(jnp.sum(m * waT[...], -1, keepdims=True) + ba[...])
        agg = agg + m * att
        mx = jax.nn.silu(jnp.dot(m, wx1[...], preferred_element_type=F32, precision=jax.lax.Precision.HIGHEST) + bx1[...])
        wv = jnp.tanh(jnp.sum(mx * wx2T[...], -1, keepdims=True) + bx2[...])
        xdelta = xdelta + d * wv
    xnew = x + xdelta / K_NEI
    hin = jnp.dot(h, wh1a[...], preferred_element_type=F32, precision=jax.lax.Precision.HIGHEST)
    hin = hin + jnp.dot(agg, wh1b[...], preferred_element_type=F32, precision=jax.lax.Precision.HIGHEST)
    hin = hin + jnp.dot(na_ref[...], wh1c[...], preferred_element_type=F32, precision=jax.lax.Precision.HIGHEST) + bh1[...]
    hnew = h + jnp.dot(jax.nn.silu(hin), wh2[...], preferred_element_type=F32, precision=jax.lax.Precision.HIGHEST) + bh2[...]
    sout_ref[...] = jnp.concatenate([hnew, xnew, jnp.zeros((BLK, DG - HID - 3), F32)], axis=-1)


def _layer_kernel(*args):
    _layer_body(False, *args)


def _layer0_kernel(s_ref, hcx_ref, na_ref, cen_ref, *rest):
    *ws, sout, eaout, nvout = rest
    _layer_body(True, s_ref, hcx_ref, None, None, na_ref, *ws,
                sout, eaout_ref=eaout, nvout_ref=nvout, cen_ref=cen_ref)


_spec_node = lambda c: pl.BlockSpec((BLK, c), lambda i: (i, 0))
_spec_nei = lambda c: pl.BlockSpec((K_NEI, BLK, c), lambda i: (0, i, 0))
_spec_nv = pl.BlockSpec((NORD, BLK, 3), lambda i: (0, i, 0))
_full = lambda s: pl.BlockSpec(s, lambda i: tuple(0 for _ in s))


def _egnn_layer(state_h, hcxT, eaT, nv5, na_h, w):
    in_specs = [
        _spec_node(DG), _spec_nei(DG), _spec_nei(RBF),
        _spec_nv, _spec_node(NODE_IN),
    ] + [_full(x.shape) for x in w]
    return pl.pallas_call(
        _layer_kernel,
        grid=(NBLK_H,),
        in_specs=in_specs,
        out_specs=_spec_node(DG),
        out_shape=jax.ShapeDtypeStruct((NH, DG), F32),
        compiler_params=pltpu.CompilerParams(dimension_semantics=("parallel",)),
    )(state_h, hcxT, eaT, nv5, na_h, *w)


def _egnn_layer0(state_h, hcxT, na_h, w):
    in_specs = [
        _spec_node(DG), _spec_nei(DG), _spec_node(NODE_IN),
        _full((1, RBF)),
    ] + [_full(x.shape) for x in w]
    return pl.pallas_call(
        _layer0_kernel,
        grid=(NBLK_H,),
        in_specs=in_specs,
        out_specs=[_spec_node(DG), _spec_nei(RBF), _spec_nv],
        out_shape=[jax.ShapeDtypeStruct((NH, DG), F32),
                   jax.ShapeDtypeStruct((K_NEI, NH, RBF), F32),
                   jax.ShapeDtypeStruct((NORD, NH, 3), F32)],
        compiler_params=pltpu.CompilerParams(dimension_semantics=("parallel",)),
    )(state_h, hcxT, na_h, _CENTERS.reshape(1, RBF), *w)


def kernel(X, S, RP, ID, Seg, center, keypoints, bid, k_bid, params):
    p = params
    X = (X - center[bid][:, None, :]) / STD
    ori_X = X[:, CA]
    kp = (keypoints - center[k_bid]) / STD
    rots, tr = _rots()
    Xb = X.reshape(B, NPC, C_ATOM, 3)
    Xab = jnp.einsum('bncd,bde->bnce', Xb[:, :N_AB], rots) + tr[:, None, None, :]
    X = jnp.concatenate([Xab, Xb[:, N_AB:]], 1).reshape(N, C_ATOM, 3)
    tkp = jnp.einsum('bkd,bde->bke', kp.reshape(B, KP, 3), rots) + tr[:, None, :]
    node_attr = jnp.concatenate([p['emb_S'][S], p['emb_RP'][RP], p['emb_Seg'][Seg] + p['emb_ID'][ID]], -1)
    Xca = X[:, CA]
    col = _knn(Xca)
    colT = col.T
    idx_h = [colT[:, h * NH:(h + 1) * NH].reshape(-1) for h in range(4)]
    init_X = Xca

    H0 = node_attr @ p['W_in'] + p['b_in']
    states = [jnp.concatenate([H0[h * NH:(h + 1) * NH],
                               Xca[h * NH:(h + 1) * NH],
                               jnp.zeros((NH, DG - HID - 3), F32)], axis=1)
              for h in range(4)]
    na_h = [node_attr[h * NH:(h + 1) * NH] for h in range(4)]
    ea_h = [None] * 4; nv_h = [None] * 4
    for l in range(L):
        w = [
            p['We1'][l][:HID], p['We1'][l][HID:2 * HID], p['We1'][l][2 * HID:2 * HID + RBF],
            p['We1'][l][2 * HID + RBF:], p['be1'][l][None, :],
            p['We2'][l], p['be2'][l][None, :],
            p['Wa'][l].reshape(1, HID), p['ba'][l].reshape(1, 1),
            p['Wx1'][l], p['bx1'][l][None, :],
            p['Wx2'][l].reshape(1, HID), p['bx2'][l].reshape(1, 1),
            p['Wh1'][l][:HID], p['Wh1'][l][HID:2 * HID], p['Wh1'][l][2 * HID:],
            p['bh1'][l][None, :], p['Wh2'][l], p['bh2'][l][None, :],
        ]
        hcx = [_gather_hx(states[h], idx_h[h]) for h in range(4)]
        for h in range(4):
            if l == 0:
                states[h], ea_h[h], nv_h[h] = _egnn_layer0(states[h], hcx[h], na_h[h], w)
            else:
                states[h] = _egnn_layer(states[h], hcx[h], ea_h[h], nv_h[h], na_h[h], w)
    H = jnp.concatenate([st[:, :HID] for st in states], axis=0)
    Xc = jnp.concatenate([st[:, HID:HID + 3] for st in states], axis=0)

    Hb = H.reshape(B, NPC, HID); Xb2 = Xc.reshape(B, NPC, 3)
    iXb = init_X.reshape(B, NPC, 3); oXb = ori_X.reshape(B, NPC, 3)
    kpb = kp.reshape(B, KP, 3)
    I3 = jnp.eye(3)
    ot = 0.0; dock = 0.0; stable = 0.0; match = 0.0; rmsd = 0.0; f_n = 0.1
    for i in range(B):
        H1 = Hb[i, :N_AB]; H2 = Hb[i, N_AB:]; X1 = Xb2[i, :N_AB]; X2 = Xb2[i, N_AB:]
        V1 = jnp.einsum('kde,e->kd', p['w1_mats'], H2.mean(0))
        A1 = jax.nn.softmax((H1 @ V1.T) / np.sqrt(HID), axis=0)
        Y1 = A1.T @ X1; YH1 = A1.T @ H1
        V2 = jnp.einsum('kde,e->kd', p['w2_mats'], H1.mean(0))
        A2 = jax.nn.softmax((H2 @ V2.T) / np.sqrt(HID), axis=0)
        Y2 = A2.T @ X2; YH2 = A2.T @ H2
        P1 = tkp[i]; P2 = kpb[i]
        mi1 = jnp.argmin(_cdist(Y1, P1), axis=1)
        ot = ot + _mse(Y1, P1[mi1])
        mi2 = jnp.argmin(_cdist(Y2, P2), axis=1)
        ot = ot + _mse(Y2, P2[mi2])
        ot = ot / 2
        R, t = _kabsch(Y1, Y2)
        dock = dock + _mse(rots[i] @ R, I3) + _mse(tr[i][None, :] @ R, -t[None, :])
        stable = stable + jax.nn.softplus(-_maxtri(Y1)) + jax.nn.softplus(-_maxtri(Y2))
        stable = stable / 2
        D12 = _cdist(P2[mi1], Y2); mi12 = jnp.argmin(D12, 1); ma12 = jnp.argmax(D12, 1)
        match = match + jnp.mean(jax.nn.softplus((1 - 2 * f_n) * jnp.sum(YH1 * YH2[ma12], -1) - jnp.sum(YH1 * YH2[mi12], -1)))
        D21 = _cdist(P1[mi2], Y1); mi21 = jnp.argmin(D21, 1); ma21 = jnp.argmax(D21, 1)
        match = match + jnp.mean(jax.nn.softplus((1 - 2 * f_n) * jnp.sum(YH2 * YH1[ma21], -1) - jnp.sum(YH2 * YH1[mi21], -1)))
        match = match / 2
        rmsd = rmsd + _mse(iXb[i, :N_AB] @ R + t, oXb[i, :N_AB])
    ot = ot / B; dock = dock / B; stable = stable / B; match = match / B; rmsd = rmsd / B
    loss = 2 * ot + dock + stable + match
    return loss, (ot, dock, stable, match, rmsd)


# speedup vs baseline: 1.0301x; 1.0301x over previous
import functools

import numpy as np
import jax
import jax.numpy as jnp
from jax.experimental import pallas as pl
from jax.experimental.pallas import tpu as pltpu
from jax.experimental.pallas import tpu_sc as plsc

B = 4; N_AB = 1024; N_AG = 1536; NPC = N_AB + N_AG; N = B * NPC
C_ATOM = 4; CA = 1; K_NEI = 9; EMB = 64; HID = 128; KP = 10; L = 4; RBF = 16
ORDERS = (2, 3, 4, 5, 6); NORD = len(ORDERS)
NODE_IN = 3 * EMB; MSG_IN = 2 * HID + RBF + NORD; STD = 10.0
_TRI = np.array([(a, b, c) for a in range(KP) for b in range(a + 1, KP) for c in range(b + 1, KP)], dtype=np.int32)
_CENTERS = jnp.linspace(0.0, 1.5, RBF)

BLK = 512
NBLK_H = (N // 2) // BLK

F32 = jnp.float32


def _mse(a, b):
    return jnp.mean((a - b) ** 2)


def _cdist(a, b):
    return jnp.sqrt(jnp.sum((a[:, None, :] - b[None, :, :]) ** 2, -1) + 1e-12)


def _rots():
    A = jax.random.normal(jax.random.key(42), (B, 3, 3))
    Q, _ = jnp.linalg.qr(A)
    d = jnp.sign(jnp.linalg.det(Q))
    Q = Q.at[:, :, 2].multiply(d[:, None])
    tr = jax.random.uniform(jax.random.key(43), (B, 3))
    return Q, tr

RBLK = 512
NRBLK = NPC // RBLK


def _knn_kernel(xr_ref, xct_ref, nbr_ref, d2_ref):
    b = pl.program_id(0)
    rb = pl.program_id(1)
    xr = xr_ref[0]
    xct = xct_ref[0]
    sqr = jnp.sum(xr * xr, -1, keepdims=True)
    sqc = jnp.sum(xct * xct, 0, keepdims=True)
    d2 = sqr + sqc - 2.0 * jnp.dot(xr, xct, preferred_element_type=F32, precision=jax.lax.Precision.HIGHEST)
    row0 = rb * RBLK
    riota = jax.lax.broadcasted_iota(jnp.int32, (RBLK, NPC), 0) + row0
    ciota = jax.lax.broadcasted_iota(jnp.int32, (RBLK, NPC), 1)
    d2_ref[...] = jnp.where(riota == ciota, d2 + 1e9, d2)
    cols = []
    for _ in range(K_NEI):
        d2m = d2_ref[...]
        mn = jnp.min(d2m, axis=1, keepdims=True)
        cand = jnp.where(d2m == mn, ciota, jnp.int32(NPC + 1))
        idx = jnp.min(cand, axis=1, keepdims=True)
        cols.append(idx)
        d2_ref[...] = jnp.where(ciota == idx, jnp.float32(jnp.inf), d2m)
    nbr_ref[...] = jnp.concatenate(cols, axis=1) + (b % 2) * NPC


def _knn(Xca):
    Xb = Xca.reshape(B, NPC, 3)
    XbT = Xb.transpose(0, 2, 1)
    col = pl.pallas_call(
        _knn_kernel,
        grid=(B, NRBLK),
        in_specs=[
            pl.BlockSpec((1, RBLK, 3), lambda b, r: (b, r, 0)),
            pl.BlockSpec((1, 3, NPC), lambda b, r: (b, 0, 0)),
        ],
        out_specs=pl.BlockSpec((RBLK, K_NEI), lambda b, r: (b * NRBLK + r, 0)),
        out_shape=jax.ShapeDtypeStruct((N, K_NEI), jnp.int32),
        scratch_shapes=[pltpu.VMEM((RBLK, NPC), F32)],
        compiler_params=pltpu.CompilerParams(dimension_semantics=("parallel", "parallel")),
    )(Xb, XbT)
    return col


NH = N // 2
E_TOT = NH * K_NEI
DG = 256
_SC = plsc.get_sparse_core_info()
NW = _SC.num_cores * _SC.num_subcores
EPW = E_TOT // NW
CH = 240
NCH = EPW // CH


@functools.partial(
    pl.kernel,
    mesh=plsc.VectorSubcoreMesh(core_axis_name="c", subcore_axis_name="s"),
    out_type=jax.ShapeDtypeStruct((E_TOT, DG), jnp.float32),
    scratch_types=[
        pltpu.VMEM((CH,), jnp.int32),
        pltpu.VMEM((CH,), jnp.int32),
        pltpu.VMEM((CH, DG), jnp.float32),
        pltpu.VMEM((CH, DG), jnp.float32),
        pltpu.SemaphoreType.DMA,
    ],
)
def _sc_gather(table_hbm, idx_hbm, out_hbm, idx0, idx1, rows0, rows1, sem):
    wid = jax.lax.axis_index("s") * _SC.num_cores + jax.lax.axis_index("c")
    base = wid * EPW
    idxb = [idx0, idx1]
    rowsb = [rows0, rows1]
    pltpu.sync_copy(idx_hbm.at[pl.ds(base, CH)], idxb[0])
    copies = [pltpu.async_copy(table_hbm.at[idxb[0]], rowsb[0], sem)]
    for i in range(NCH):
        if i + 1 < NCH:
            pltpu.sync_copy(idx_hbm.at[pl.ds(base + (i + 1) * CH, CH)], idxb[(i + 1) % 2])
            copies.append(pltpu.async_copy(table_hbm.at[idxb[(i + 1) % 2]], rowsb[(i + 1) % 2], sem))
        copies[i].wait()
        pltpu.sync_copy(rowsb[i % 2], out_hbm.at[pl.ds(base + i * CH, CH)])


def _gather_hx(state_h, idx_h):
    return _sc_gather(state_h, idx_h).reshape(K_NEI, NH, DG)


def _kabsch(Y1, Y2):
    c1 = Y1.mean(0); c2 = Y2.mean(0)
    Hm = (Y1 - c1).T @ (Y2 - c2)
    U, _, Vt = jnp.linalg.svd(Hm, full_matrices=False)
    d = jnp.sign(jnp.linalg.det(U @ Vt))
    D = jnp.diag(jnp.concatenate([jnp.ones(2), d[None]]))
    R = U @ D @ Vt
    t = c2 - c1 @ R
    return R, t


def _maxtri(Y):
    a = Y[_TRI[:, 0]]; b = Y[_TRI[:, 1]]; c = Y[_TRI[:, 2]]
    return jnp.max(0.5 * jnp.linalg.norm(jnp.cross(b - a, c - a), axis=-1))


def _layer_body(first, s_ref, hcx_ref, ea_ref, nv_ref, na_ref,
                we1a, we1b, we1c, we1d, be1, we2, be2, waT, ba,
                wx1, bx1, wx2T, bx2, wh1a, wh1b, wh1c, bh1, wh2, bh2,
                sout_ref, eaout_ref=None, nvout_ref=None, cen_ref=None):
    sblk = s_ref[...]
    h = sblk[:, :HID]
    x = sblk[:, HID:HID + 3]
    if first:
        ea_js = []
        fo = [jnp.zeros((BLK, 3), F32) for _ in ORDERS]
        for j in range(K_NEI):
            xc = hcx_ref[j][:, HID:HID + 3]
            dj = x - xc
            d0 = jnp.sqrt(jnp.sum(dj * dj, -1, keepdims=True))
            ea_j = jnp.exp(-((d0 - cen_ref[...]) ** 2) / (2 * 0.1 ** 2))
            ea_js.append(ea_j)
            eaout_ref[j] = ea_j
            dist = d0 + 1e-8
            for o in range(NORD):
                fo[o] = fo[o] + dj / dist ** ORDERS[o]
        nv_js = []
        for o in range(NORD):
            f = fo[o]
            nv_o = f / (jnp.sqrt(jnp.sum(f * f, -1, keepdims=True)) + 1e-8)
            nv_js.append(nv_o)
            nvout_ref[o] = nv_o
    base = jnp.dot(h, we1a[...], preferred_element_type=F32, precision=jax.lax.Precision.HIGHEST) + be1[...]
    agg = jnp.zeros((BLK, HID), F32)
    xdelta = jnp.zeros((BLK, 3), F32)
    for j in range(K_NEI):
        hcx = hcx_ref[j]
        hc = hcx[:, :HID]
        xc = hcx[:, HID:HID + 3]
        d = x - xc
        dist = jnp.sqrt(jnp.sum(d * d, -1, keepdims=True)) + 1e-8
        u = d / dist
        ea_j = ea_js[j] if first else ea_ref[j]
        mpre = base + jnp.dot(hc, we1b[...], preferred_element_type=F32, precision=jax.lax.Precision.HIGHEST)
        mpre = mpre + jnp.dot(ea_j, we1c[...], preferred_element_type=F32, precision=jax.lax.Precision.HIGHEST)
        for o in range(NORD):
            nv_o = nv_js[o] if first else nv_ref[o]
            nf_o = jnp.sum(nv_o * u, -1, keepdims=True)
            mpre = mpre + nf_o * we1d[o:o + 1, :]
        m = jax.nn.silu(mpre)
        m = jax.nn.silu(jnp.dot(m, we2[...], preferred_element_type=F32, precision=jax.lax.Precision.HIGHEST) + be2[...])
        att = jax.nn.sigmoid(jnp.sum(m * waT[...], -1, keepdims=True) + ba[...])
        agg = agg + m * att
        mx = jax.nn.silu(jnp.dot(m, wx1[...], preferred_element_type=F32, precision=jax.lax.Precision.HIGHEST) + bx1[...])
        wv = jnp.tanh(jnp.sum(mx * wx2T[...], -1, keepdims=True) + bx2[...])
        xdelta = xdelta + d * wv
    xnew = x + xdelta / K_NEI
    hin = jnp.dot(h, wh1a[...], preferred_element_type=F32, precision=jax.lax.Precision.HIGHEST)
    hin = hin + jnp.dot(agg, wh1b[...], preferred_element_type=F32, precision=jax.lax.Precision.HIGHEST)
    hin = hin + jnp.dot(na_ref[...], wh1c[...], preferred_element_type=F32, precision=jax.lax.Precision.HIGHEST) + bh1[...]
    hnew = h + jnp.dot(jax.nn.silu(hin), wh2[...], preferred_element_type=F32, precision=jax.lax.Precision.HIGHEST) + bh2[...]
    sout_ref[...] = jnp.concatenate([hnew, xnew, jnp.zeros((BLK, DG - HID - 3), F32)], axis=-1)


def _layer_kernel(*args):
    _layer_body(False, *args)


def _layer0_kernel(s_ref, hcx_ref, na_ref, cen_ref, *rest):
    *ws, sout, eaout, nvout = rest
    _layer_body(True, s_ref, hcx_ref, None, None, na_ref, *ws,
                sout, eaout_ref=eaout, nvout_ref=nvout, cen_ref=cen_ref)


_spec_node = lambda c: pl.BlockSpec((BLK, c), lambda i: (i, 0))
_spec_nei = lambda c: pl.BlockSpec((K_NEI, BLK, c), lambda i: (0, i, 0))
_spec_nv = pl.BlockSpec((NORD, BLK, 3), lambda i: (0, i, 0))
_full = lambda s: pl.BlockSpec(s, lambda i: tuple(0 for _ in s))


def _egnn_layer(state_h, hcxT, eaT, nv5, na_h, w):
    in_specs = [
        _spec_node(DG), _spec_nei(DG), _spec_nei(RBF),
        _spec_nv, _spec_node(NODE_IN),
    ] + [_full(x.shape) for x in w]
    return pl.pallas_call(
        _layer_kernel,
        grid=(NBLK_H,),
        in_specs=in_specs,
        out_specs=_spec_node(DG),
        out_shape=jax.ShapeDtypeStruct((NH, DG), F32),
        compiler_params=pltpu.CompilerParams(dimension_semantics=("parallel",)),
    )(state_h, hcxT, eaT, nv5, na_h, *w)


def _egnn_layer0(state_h, hcxT, na_h, w):
    in_specs = [
        _spec_node(DG), _spec_nei(DG), _spec_node(NODE_IN),
        _full((1, RBF)),
    ] + [_full(x.shape) for x in w]
    return pl.pallas_call(
        _layer0_kernel,
        grid=(NBLK_H,),
        in_specs=in_specs,
        out_specs=[_spec_node(DG), _spec_nei(RBF), _spec_nv],
        out_shape=[jax.ShapeDtypeStruct((NH, DG), F32),
                   jax.ShapeDtypeStruct((K_NEI, NH, RBF), F32),
                   jax.ShapeDtypeStruct((NORD, NH, 3), F32)],
        compiler_params=pltpu.CompilerParams(dimension_semantics=("parallel",)),
    )(state_h, hcxT, na_h, _CENTERS.reshape(1, RBF), *w)


def kernel(X, S, RP, ID, Seg, center, keypoints, bid, k_bid, params):
    p = params
    X = (X - center[bid][:, None, :]) / STD
    ori_X = X[:, CA]
    kp = (keypoints - center[k_bid]) / STD
    rots, tr = _rots()
    Xb = X.reshape(B, NPC, C_ATOM, 3)
    Xab = jnp.einsum('bncd,bde->bnce', Xb[:, :N_AB], rots) + tr[:, None, None, :]
    X = jnp.concatenate([Xab, Xb[:, N_AB:]], 1).reshape(N, C_ATOM, 3)
    tkp = jnp.einsum('bkd,bde->bke', kp.reshape(B, KP, 3), rots) + tr[:, None, :]
    node_attr = jnp.concatenate([p['emb_S'][S], p['emb_RP'][RP], p['emb_Seg'][Seg] + p['emb_ID'][ID]], -1)
    Xca = X[:, CA]
    col = _knn(Xca)
    colT = col.T
    idx_h = [colT[:, h * NH:(h + 1) * NH].reshape(-1) for h in range(2)]
    init_X = Xca

    H0 = node_attr @ p['W_in'] + p['b_in']
    states = [jnp.concatenate([H0[h * NH:(h + 1) * NH],
                               Xca[h * NH:(h + 1) * NH],
                               jnp.zeros((NH, DG - HID - 3), F32)], axis=1)
              for h in range(2)]
    na_h = [node_attr[h * NH:(h + 1) * NH] for h in range(2)]
    ea_h = [None, None]; nv_h = [None, None]
    for l in range(L):
        w = [
            p['We1'][l][:HID], p['We1'][l][HID:2 * HID], p['We1'][l][2 * HID:2 * HID + RBF],
            p['We1'][l][2 * HID + RBF:], p['be1'][l][None, :],
            p['We2'][l], p['be2'][l][None, :],
            p['Wa'][l].reshape(1, HID), p['ba'][l].reshape(1, 1),
            p['Wx1'][l], p['bx1'][l][None, :],
            p['Wx2'][l].reshape(1, HID), p['bx2'][l].reshape(1, 1),
            p['Wh1'][l][:HID], p['Wh1'][l][HID:2 * HID], p['Wh1'][l][2 * HID:],
            p['bh1'][l][None, :], p['Wh2'][l], p['bh2'][l][None, :],
        ]
        hcx = [_gather_hx(states[h], idx_h[h]) for h in range(2)]
        for h in range(2):
            if l == 0:
                states[h], ea_h[h], nv_h[h] = _egnn_layer0(states[h], hcx[h], na_h[h], w)
            else:
                states[h] = _egnn_layer(states[h], hcx[h], ea_h[h], nv_h[h], na_h[h], w)
    H = jnp.concatenate([states[0][:, :HID], states[1][:, :HID]], axis=0)
    Xc = jnp.concatenate([states[0][:, HID:HID + 3], states[1][:, HID:HID + 3]], axis=0)

    Hb = H.reshape(B, NPC, HID); Xb2 = Xc.reshape(B, NPC, 3)
    iXb = init_X.reshape(B, NPC, 3); oXb = ori_X.reshape(B, NPC, 3)
    kpb = kp.reshape(B, KP, 3)
    I3 = jnp.eye(3)
    ot = 0.0; dock = 0.0; stable = 0.0; match = 0.0; rmsd = 0.0; f_n = 0.1
    for i in range(B):
        H1 = Hb[i, :N_AB]; H2 = Hb[i, N_AB:]; X1 = Xb2[i, :N_AB]; X2 = Xb2[i, N_AB:]
        V1 = jnp.einsum('kde,e->kd', p['w1_mats'], H2.mean(0))
        A1 = jax.nn.softmax((H1 @ V1.T) / np.sqrt(HID), axis=0)
        Y1 = A1.T @ X1; YH1 = A1.T @ H1
        V2 = jnp.einsum('kde,e->kd', p['w2_mats'], H1.mean(0))
        A2 = jax.nn.softmax((H2 @ V2.T) / np.sqrt(HID), axis=0)
        Y2 = A2.T @ X2; YH2 = A2.T @ H2
        P1 = tkp[i]; P2 = kpb[i]
        mi1 = jnp.argmin(_cdist(Y1, P1), axis=1)
        ot = ot + _mse(Y1, P1[mi1])
        mi2 = jnp.argmin(_cdist(Y2, P2), axis=1)
        ot = ot + _mse(Y2, P2[mi2])
        ot = ot / 2
        R, t = _kabsch(Y1, Y2)
        dock = dock + _mse(rots[i] @ R, I3) + _mse(tr[i][None, :] @ R, -t[None, :])
        stable = stable + jax.nn.softplus(-_maxtri(Y1)) + jax.nn.softplus(-_maxtri(Y2))
        stable = stable / 2
        D12 = _cdist(P2[mi1], Y2); mi12 = jnp.argmin(D12, 1); ma12 = jnp.argmax(D12, 1)
        match = match + jnp.mean(jax.nn.softplus((1 - 2 * f_n) * jnp.sum(YH1 * YH2[ma12], -1) - jnp.sum(YH1 * YH2[mi12], -1)))
        D21 = _cdist(P1[mi2], Y1); mi21 = jnp.argmin(D21, 1); ma21 = jnp.argmax(D21, 1)
        match = match + jnp.mean(jax.nn.softplus((1 - 2 * f_n) * jnp.sum(YH2 * YH1[ma21], -1) - jnp.sum(YH2 * YH1[mi21], -1)))
        match = match / 2
        rmsd = rmsd + _mse(iXb[i, :N_AB] @ R + t, oXb[i, :N_AB])
    ot = ot / B; dock = dock / B; stable = stable / B; match = match / B; rmsd = rmsd / B
    loss = 2 * ot + dock + stable + match
    return loss, (ot, dock, stable, match, rmsd)


# speedup vs baseline: 1.0329x; 1.0027x over previous
import functools

import numpy as np
import jax
import jax.numpy as jnp
from jax.experimental import pallas as pl
from jax.experimental.pallas import tpu as pltpu
from jax.experimental.pallas import tpu_sc as plsc

B = 4; N_AB = 1024; N_AG = 1536; NPC = N_AB + N_AG; N = B * NPC
C_ATOM = 4; CA = 1; K_NEI = 9; EMB = 64; HID = 128; KP = 10; L = 4; RBF = 16
ORDERS = (2, 3, 4, 5, 6); NORD = len(ORDERS)
NODE_IN = 3 * EMB; MSG_IN = 2 * HID + RBF + NORD; STD = 10.0
_TRI = np.array([(a, b, c) for a in range(KP) for b in range(a + 1, KP) for c in range(b + 1, KP)], dtype=np.int32)
_CENTERS = jnp.linspace(0.0, 1.5, RBF)

BLK = 640
NBLK_H = (N // 2) // BLK

F32 = jnp.float32


def _mse(a, b):
    return jnp.mean((a - b) ** 2)


def _cdist(a, b):
    return jnp.sqrt(jnp.sum((a[:, None, :] - b[None, :, :]) ** 2, -1) + 1e-12)


def _rots():
    A = jax.random.normal(jax.random.key(42), (B, 3, 3))
    Q, _ = jnp.linalg.qr(A)
    d = jnp.sign(jnp.linalg.det(Q))
    Q = Q.at[:, :, 2].multiply(d[:, None])
    tr = jax.random.uniform(jax.random.key(43), (B, 3))
    return Q, tr

RBLK = 512
NRBLK = NPC // RBLK


def _knn_kernel(xr_ref, xct_ref, nbr_ref, d2_ref):
    b = pl.program_id(0)
    rb = pl.program_id(1)
    xr = xr_ref[0]
    xct = xct_ref[0]
    sqr = jnp.sum(xr * xr, -1, keepdims=True)
    sqc = jnp.sum(xct * xct, 0, keepdims=True)
    d2 = sqr + sqc - 2.0 * jnp.dot(xr, xct, preferred_element_type=F32, precision=jax.lax.Precision.HIGHEST)
    row0 = rb * RBLK
    riota = jax.lax.broadcasted_iota(jnp.int32, (RBLK, NPC), 0) + row0
    ciota = jax.lax.broadcasted_iota(jnp.int32, (RBLK, NPC), 1)
    d2_ref[...] = jnp.where(riota == ciota, d2 + 1e9, d2)
    cols = []
    for _ in range(K_NEI):
        d2m = d2_ref[...]
        mn = jnp.min(d2m, axis=1, keepdims=True)
        cand = jnp.where(d2m == mn, ciota, jnp.int32(NPC + 1))
        idx = jnp.min(cand, axis=1, keepdims=True)
        cols.append(idx)
        d2_ref[...] = jnp.where(ciota == idx, jnp.float32(jnp.inf), d2m)
    nbr_ref[...] = jnp.concatenate(cols, axis=1) + (b % 2) * NPC


def _knn(Xca):
    Xb = Xca.reshape(B, NPC, 3)
    XbT = Xb.transpose(0, 2, 1)
    col = pl.pallas_call(
        _knn_kernel,
        grid=(B, NRBLK),
        in_specs=[
            pl.BlockSpec((1, RBLK, 3), lambda b, r: (b, r, 0)),
            pl.BlockSpec((1, 3, NPC), lambda b, r: (b, 0, 0)),
        ],
        out_specs=pl.BlockSpec((RBLK, K_NEI), lambda b, r: (b * NRBLK + r, 0)),
        out_shape=jax.ShapeDtypeStruct((N, K_NEI), jnp.int32),
        scratch_shapes=[pltpu.VMEM((RBLK, NPC), F32)],
        compiler_params=pltpu.CompilerParams(dimension_semantics=("parallel", "parallel")),
    )(Xb, XbT)
    return col


NH = N // 2
E_TOT = NH * K_NEI
DG = 256
_SC = plsc.get_sparse_core_info()
NW = _SC.num_cores * _SC.num_subcores
EPW = E_TOT // NW
CH = 240
NCH = EPW // CH


@functools.partial(
    pl.kernel,
    mesh=plsc.VectorSubcoreMesh(core_axis_name="c", subcore_axis_name="s"),
    out_type=jax.ShapeDtypeStruct((E_TOT, DG), jnp.float32),
    scratch_types=[
        pltpu.VMEM((CH,), jnp.int32),
        pltpu.VMEM((CH,), jnp.int32),
        pltpu.VMEM((CH, DG), jnp.float32),
        pltpu.VMEM((CH, DG), jnp.float32),
        pltpu.SemaphoreType.DMA,
    ],
)
def _sc_gather(table_hbm, idx_hbm, out_hbm, idx0, idx1, rows0, rows1, sem):
    wid = jax.lax.axis_index("s") * _SC.num_cores + jax.lax.axis_index("c")
    base = wid * EPW
    idxb = [idx0, idx1]
    rowsb = [rows0, rows1]
    pltpu.sync_copy(idx_hbm.at[pl.ds(base, CH)], idxb[0])
    copies = [pltpu.async_copy(table_hbm.at[idxb[0]], rowsb[0], sem)]
    for i in range(NCH):
        if i + 1 < NCH:
            pltpu.sync_copy(idx_hbm.at[pl.ds(base + (i + 1) * CH, CH)], idxb[(i + 1) % 2])
            copies.append(pltpu.async_copy(table_hbm.at[idxb[(i + 1) % 2]], rowsb[(i + 1) % 2], sem))
        copies[i].wait()
        pltpu.sync_copy(rowsb[i % 2], out_hbm.at[pl.ds(base + i * CH, CH)])


def _gather_hx(state_h, idx_h):
    return _sc_gather(state_h, idx_h).reshape(K_NEI, NH, DG)


def _kabsch(Y1, Y2):
    c1 = Y1.mean(0); c2 = Y2.mean(0)
    Hm = (Y1 - c1).T @ (Y2 - c2)
    U, _, Vt = jnp.linalg.svd(Hm, full_matrices=False)
    d = jnp.sign(jnp.linalg.det(U @ Vt))
    D = jnp.diag(jnp.concatenate([jnp.ones(2), d[None]]))
    R = U @ D @ Vt
    t = c2 - c1 @ R
    return R, t


def _maxtri(Y):
    a = Y[_TRI[:, 0]]; b = Y[_TRI[:, 1]]; c = Y[_TRI[:, 2]]
    return jnp.max(0.5 * jnp.linalg.norm(jnp.cross(b - a, c - a), axis=-1))


def _layer_body(first, s_ref, hcx_ref, ea_ref, nv_ref, na_ref,
                we1a, we1b, we1c, we1d, be1, we2, be2, waT, ba,
                wx1, bx1, wx2T, bx2, wh1a, wh1b, wh1c, bh1, wh2, bh2,
                sout_ref, eaout_ref=None, nvout_ref=None, cen_ref=None):
    sblk = s_ref[...]
    h = sblk[:, :HID]
    x = sblk[:, HID:HID + 3]
    if first:
        ea_js = []
        fo = [jnp.zeros((BLK, 3), F32) for _ in ORDERS]
        for j in range(K_NEI):
            xc = hcx_ref[j][:, HID:HID + 3]
            dj = x - xc
            d0 = jnp.sqrt(jnp.sum(dj * dj, -1, keepdims=True))
            ea_j = jnp.exp(-((d0 - cen_ref[...]) ** 2) / (2 * 0.1 ** 2))
            ea_js.append(ea_j)
            eaout_ref[j] = ea_j
            dist = d0 + 1e-8
            for o in range(NORD):
                fo[o] = fo[o] + dj / dist ** ORDERS[o]
        nv_js = []
        for o in range(NORD):
            f = fo[o]
            nv_o = f / (jnp.sqrt(jnp.sum(f * f, -1, keepdims=True)) + 1e-8)
            nv_js.append(nv_o)
            nvout_ref[o] = nv_o
    base = jnp.dot(h, we1a[...], preferred_element_type=F32, precision=jax.lax.Precision.HIGHEST) + be1[...]
    agg = jnp.zeros((BLK, HID), F32)
    xdelta = jnp.zeros((BLK, 3), F32)
    for j in range(K_NEI):
        hcx = hcx_ref[j]
        hc = hcx[:, :HID]
        xc = hcx[:, HID:HID + 3]
        d = x - xc
        dist = jnp.sqrt(jnp.sum(d * d, -1, keepdims=True)) + 1e-8
        u = d / dist
        ea_j = ea_js[j] if first else ea_ref[j]
        mpre = base + jnp.dot(hc, we1b[...], preferred_element_type=F32, precision=jax.lax.Precision.HIGHEST)
        mpre = mpre + jnp.dot(ea_j, we1c[...], preferred_element_type=F32, precision=jax.lax.Precision.HIGHEST)
        for o in range(NORD):
            nv_o = nv_js[o] if first else nv_ref[o]
            nf_o = jnp.sum(nv_o * u, -1, keepdims=True)
            mpre = mpre + nf_o * we1d[o:o + 1, :]
        m = jax.nn.silu(mpre)
        m = jax.nn.silu(jnp.dot(m, we2[...], preferred_element_type=F32, precision=jax.lax.Precision.HIGHEST) + be2[...])
        att = jax.nn.sigmoid(jnp.sum(m * waT[...], -1, keepdims=True) + ba[...])
        agg = agg + m * att
        mx = jax.nn.silu(jnp.dot(m, wx1[...], preferred_element_type=F32, precision=jax.lax.Precision.HIGHEST) + bx1[...])
        wv = jnp.tanh(jnp.sum(mx * wx2T[...], -1, keepdims=True) + bx2[...])
        xdelta = xdelta + d * wv
    xnew = x + xdelta / K_NEI
    hin = jnp.dot(h, wh1a[...], preferred_element_type=F32, precision=jax.lax.Precision.HIGHEST)
    hin = hin + jnp.dot(agg, wh1b[...], preferred_element_type=F32, precision=jax.lax.Precision.HIGHEST)
    hin = hin + jnp.dot(na_ref[...], wh1c[...], preferred_element_type=F32, precision=jax.lax.Precision.HIGHEST) + bh1[...]
    hnew = h + jnp.dot(jax.nn.silu(hin), wh2[...], preferred_element_type=F32, precision=jax.lax.Precision.HIGHEST) + bh2[...]
    sout_ref[...] = jnp.concatenate([hnew, xnew, jnp.zeros((BLK, DG - HID - 3), F32)], axis=-1)


def _layer_kernel(*args):
    _layer_body(False, *args)


def _layer0_kernel(s_ref, hcx_ref, na_ref, cen_ref, *rest):
    *ws, sout, eaout, nvout = rest
    _layer_body(True, s_ref, hcx_ref, None, None, na_ref, *ws,
                sout, eaout_ref=eaout, nvout_ref=nvout, cen_ref=cen_ref)


_spec_node = lambda c: pl.BlockSpec((BLK, c), lambda i: (i, 0))
_spec_nei = lambda c: pl.BlockSpec((K_NEI, BLK, c), lambda i: (0, i, 0))
_spec_nv = pl.BlockSpec((NORD, BLK, 3), lambda i: (0, i, 0))
_full = lambda s: pl.BlockSpec(s, lambda i: tuple(0 for _ in s))


def _egnn_layer(state_h, hcxT, eaT, nv5, na_h, w):
    in_specs = [
        _spec_node(DG), _spec_nei(DG), _spec_nei(RBF),
        _spec_nv, _spec_node(NODE_IN),
    ] + [_full(x.shape) for x in w]
    return pl.pallas_call(
        _layer_kernel,
        grid=(NBLK_H,),
        in_specs=in_specs,
        out_specs=_spec_node(DG),
        out_shape=jax.ShapeDtypeStruct((NH, DG), F32),
        compiler_params=pltpu.CompilerParams(dimension_semantics=("parallel",)),
    )(state_h, hcxT, eaT, nv5, na_h, *w)


def _egnn_layer0(state_h, hcxT, na_h, w):
    in_specs = [
        _spec_node(DG), _spec_nei(DG), _spec_node(NODE_IN),
        _full((1, RBF)),
    ] + [_full(x.shape) for x in w]
    return pl.pallas_call(
        _layer0_kernel,
        grid=(NBLK_H,),
        in_specs=in_specs,
        out_specs=[_spec_node(DG), _spec_nei(RBF), _spec_nv],
        out_shape=[jax.ShapeDtypeStruct((NH, DG), F32),
                   jax.ShapeDtypeStruct((K_NEI, NH, RBF), F32),
                   jax.ShapeDtypeStruct((NORD, NH, 3), F32)],
        compiler_params=pltpu.CompilerParams(dimension_semantics=("parallel",)),
    )(state_h, hcxT, na_h, _CENTERS.reshape(1, RBF), *w)


def kernel(X, S, RP, ID, Seg, center, keypoints, bid, k_bid, params):
    p = params
    X = (X - center[bid][:, None, :]) / STD
    ori_X = X[:, CA]
    kp = (keypoints - center[k_bid]) / STD
    rots, tr = _rots()
    Xb = X.reshape(B, NPC, C_ATOM, 3)
    Xab = jnp.einsum('bncd,bde->bnce', Xb[:, :N_AB], rots) + tr[:, None, None, :]
    X = jnp.concatenate([Xab, Xb[:, N_AB:]], 1).reshape(N, C_ATOM, 3)
    tkp = jnp.einsum('bkd,bde->bke', kp.reshape(B, KP, 3), rots) + tr[:, None, :]
    node_attr = jnp.concatenate([p['emb_S'][S], p['emb_RP'][RP], p['emb_Seg'][Seg] + p['emb_ID'][ID]], -1)
    Xca = X[:, CA]
    col = _knn(Xca)
    colT = col.T
    idx_h = [colT[:, h * NH:(h + 1) * NH].reshape(-1) for h in range(2)]
    init_X = Xca

    H0 = node_attr @ p['W_in'] + p['b_in']
    states = [jnp.concatenate([H0[h * NH:(h + 1) * NH],
                               Xca[h * NH:(h + 1) * NH],
                               jnp.zeros((NH, DG - HID - 3), F32)], axis=1)
              for h in range(2)]
    na_h = [node_attr[h * NH:(h + 1) * NH] for h in range(2)]
    ea_h = [None, None]; nv_h = [None, None]
    for l in range(L):
        w = [
            p['We1'][l][:HID], p['We1'][l][HID:2 * HID], p['We1'][l][2 * HID:2 * HID + RBF],
            p['We1'][l][2 * HID + RBF:], p['be1'][l][None, :],
            p['We2'][l], p['be2'][l][None, :],
            p['Wa'][l].reshape(1, HID), p['ba'][l].reshape(1, 1),
            p['Wx1'][l], p['bx1'][l][None, :],
            p['Wx2'][l].reshape(1, HID), p['bx2'][l].reshape(1, 1),
            p['Wh1'][l][:HID], p['Wh1'][l][HID:2 * HID], p['Wh1'][l][2 * HID:],
            p['bh1'][l][None, :], p['Wh2'][l], p['bh2'][l][None, :],
        ]
        hcx = [_gather_hx(states[h], idx_h[h]) for h in range(2)]
        for h in range(2):
            if l == 0:
                states[h], ea_h[h], nv_h[h] = _egnn_layer0(states[h], hcx[h], na_h[h], w)
            else:
                states[h] = _egnn_layer(states[h], hcx[h], ea_h[h], nv_h[h], na_h[h], w)
    H = jnp.concatenate([states[0][:, :HID], states[1][:, :HID]], axis=0)
    Xc = jnp.concatenate([states[0][:, HID:HID + 3], states[1][:, HID:HID + 3]], axis=0)

    Hb = H.reshape(B, NPC, HID); Xb2 = Xc.reshape(B, NPC, 3)
    iXb = init_X.reshape(B, NPC, 3); oXb = ori_X.reshape(B, NPC, 3)
    kpb = kp.reshape(B, KP, 3)
    I3 = jnp.eye(3)
    ot = 0.0; dock = 0.0; stable = 0.0; match = 0.0; rmsd = 0.0; f_n = 0.1
    for i in range(B):
        H1 = Hb[i, :N_AB]; H2 = Hb[i, N_AB:]; X1 = Xb2[i, :N_AB]; X2 = Xb2[i, N_AB:]
        V1 = jnp.einsum('kde,e->kd', p['w1_mats'], H2.mean(0))
        A1 = jax.nn.softmax((H1 @ V1.T) / np.sqrt(HID), axis=0)
        Y1 = A1.T @ X1; YH1 = A1.T @ H1
        V2 = jnp.einsum('kde,e->kd', p['w2_mats'], H1.mean(0))
        A2 = jax.nn.softmax((H2 @ V2.T) / np.sqrt(HID), axis=0)
        Y2 = A2.T @ X2; YH2 = A2.T @ H2
        P1 = tkp[i]; P2 = kpb[i]
        mi1 = jnp.argmin(_cdist(Y1, P1), axis=1)
        ot = ot + _mse(Y1, P1[mi1])
        mi2 = jnp.argmin(_cdist(Y2, P2), axis=1)
        ot = ot + _mse(Y2, P2[mi2])
        ot = ot / 2
        R, t = _kabsch(Y1, Y2)
        dock = dock + _mse(rots[i] @ R, I3) + _mse(tr[i][None, :] @ R, -t[None, :])
        stable = stable + jax.nn.softplus(-_maxtri(Y1)) + jax.nn.softplus(-_maxtri(Y2))
        stable = stable / 2
        D12 = _cdist(P2[mi1], Y2); mi12 = jnp.argmin(D12, 1); ma12 = jnp.argmax(D12, 1)
        match = match + jnp.mean(jax.nn.softplus((1 - 2 * f_n) * jnp.sum(YH1 * YH2[ma12], -1) - jnp.sum(YH1 * YH2[mi12], -1)))
        D21 = _cdist(P1[mi2], Y1); mi21 = jnp.argmin(D21, 1); ma21 = jnp.argmax(D21, 1)
        match = match + jnp.mean(jax.nn.softplus((1 - 2 * f_n) * jnp.sum(YH2 * YH1[ma21], -1) - jnp.sum(YH2 * YH1[mi21], -1)))
        match = match / 2
        rmsd = rmsd + _mse(iXb[i, :N_AB] @ R + t, oXb[i, :N_AB])
    ot = ot / B; dock = dock / B; stable = stable / B; match = match / B; rmsd = rmsd / B
    loss = 2 * ot + dock + stable + match
    return loss, (ot, dock, stable, match, rmsd)
